# baseline (speedup 1.0000x reference)
import functools
import math

import jax
import jax.numpy as jnp
from jax import lax
from jax.experimental import pallas as pl
from jax.experimental.pallas import tpu as pltpu
from jax.experimental.pallas import tpu_sc as plsc

D = 16
NC = 2
NS = 16
NW = NC * NS
CHUNK = 1280
GSUB = 128
BB = CHUNK // 128
NSUB = CHUNK // GSUB
NGRP = CHUNK // D
LOGC = -0.5 * D * math.log(2.0 * math.pi)


def _mu_to_rowmajor(muT):
    n = muT.shape[1]
    s_blk = 12800
    r_blk = s_blk * D // 128
    grid = pl.cdiv(n, s_blk)

    def body(x_ref, o_ref):
        t = x_ref[...].T.reshape(r_blk, 8, D)
        for c in range(8):
            o_ref[:, c * D:(c + 1) * D] = t[:, c, :]

    out = pl.pallas_call(
        body,
        grid=(grid,),
        in_specs=[pl.BlockSpec((D, s_blk), lambda i: (0, i))],
        out_specs=pl.BlockSpec((r_blk, 128), lambda i: (i, 0)),
        out_shape=jax.ShapeDtypeStruct((n * D // 128, 128), jnp.float32),
    )(muT)
    return out.reshape(n, D)


@functools.partial(jax.jit, static_argnames=("m",))
def _log_prob_sc(zq, idx2d, mu, m):
    per_w = m // NW
    n_chunks = per_w // CHUNK

    mesh = plsc.VectorSubcoreMesh(core_axis_name="c", subcore_axis_name="s")

    @functools.partial(
        pl.kernel,
        out_type=jax.ShapeDtypeStruct((m,), jnp.float32),
        mesh=mesh,
        scratch_types=[
            pltpu.VMEM((2, CHUNK), jnp.int32),
            pltpu.VMEM((2, CHUNK, D), jnp.float32),
            pltpu.VMEM((2, 2, BB, 8, 128), jnp.float32),
            pltpu.VMEM((2, CHUNK), jnp.float32),
            pltpu.SemaphoreType.DMA,
            pltpu.SemaphoreType.DMA,
            pltpu.SemaphoreType.DMA,
            pltpu.SemaphoreType.DMA,
        ],
        compiler_params=pltpu.CompilerParams(
            use_tc_tiling_on_sc=False,
            needs_layout_passes=False,
        ),
    )
    def k(zq_hbm, idx_hbm, mu_hbm, out_hbm,
          idx_v, mu_v, z_v, out_v, sem_mu, sem_z, sem_i, sem_o):
        wid = lax.axis_index("s") * NC + lax.axis_index("c")
        row_base = wid * per_w

        def fire_idx(ci, b):
            cbase = row_base + ci * CHUNK
            pltpu.async_copy(
                idx_hbm.at[pl.ds(cbase, CHUNK)], idx_v.at[b], sem_i)

        def fire_inputs(ci, b):
            cbase = row_base + ci * CHUNK
            bb0 = cbase // 128
            for g in range(NSUB):
                pltpu.async_copy(
                    mu_hbm.at[idx_v.at[b, pl.ds(g * GSUB, GSUB)]],
                    mu_v.at[b, pl.ds(g * GSUB, GSUB)],
                    sem_mu,
                )
            for fg in range(2):
                pltpu.async_copy(
                    zq_hbm.at[fg, pl.ds(bb0, BB)], z_v.at[b, fg], sem_z)

        def wait_idx(b):
            pltpu.make_async_copy(
                idx_hbm.at[pl.ds(0, CHUNK)], idx_v.at[b], sem_i).wait()

        def wait_inputs(b):
            pltpu.make_async_copy(
                mu_hbm.at[pl.ds(0, CHUNK)], mu_v.at[b], sem_mu).wait()
            pltpu.make_async_copy(
                zq_hbm.at[:, pl.ds(0, BB)], z_v.at[b], sem_z).wait()

        def fire_out(ci, b):
            cbase = row_base + ci * CHUNK
            pltpu.async_copy(
                out_v.at[b], out_hbm.at[pl.ds(cbase, CHUNK)], sem_o)

        def wait_out(b):
            pltpu.make_async_copy(
                out_v.at[b], out_hbm.at[pl.ds(0, CHUNK)], sem_o).wait()

        def compute(b):
            lane = lax.iota(jnp.int32, D)

            @plsc.parallel_loop(0, NGRP, 1, unroll=4)
            def group_body(g):
                bbl = g // 8
                b0 = (g % 8) * D
                rows = g * D + lane
                acc = []
                for d in range(D):
                    zc = z_v[b, d // 8, bbl, d % 8, pl.ds(b0, D)]
                    mc = plsc.load_gather(
                        mu_v.at[b], [rows, jnp.full((D,), d, jnp.int32)]
                    )
                    t = zc - mc
                    sq = t * t
                    if d < 4:
                        acc.append(sq)
                    else:
                        acc[d % 4] = acc[d % 4] + sq
                tot = (acc[0] + acc[1]) + (acc[2] + acc[3])
                out_v[b, pl.ds(g * D, D)] = -0.5 * tot + LOGC

        fire_idx(0, 0)
        fire_idx(1, 1)
        wait_idx(0)
        fire_inputs(0, 0)

        @pl.loop(0, n_chunks, step=2)
        def chunk_loop(cc):
            for b in range(2):
                ci = cc + b
                nb = 1 - b

                @pl.when(ci + 1 < n_chunks)
                def _():
                    wait_idx(nb)
                    fire_inputs(ci + 1, nb)

                wait_inputs(b)

                @pl.when(ci + 2 < n_chunks)
                def _():
                    fire_idx(ci + 2, b)

                @pl.when(ci >= 2)
                def _():
                    wait_out(b)

                compute(b)
                fire_out(ci, b)

        wait_out(0)
        wait_out(1)

    return k(zq, idx2d, mu)


def kernel(z, sensor_idx, mu):
    m = z.shape[0]
    zq = z.reshape(m // 128, 128, 2, 8).transpose(2, 0, 3, 1)
    idx2d = sensor_idx.astype(jnp.int32)
    mu_rm = _mu_to_rowmajor(mu.T)
    return _log_prob_sc(zq, idx2d, mu_rm, m)


# speedup vs baseline: 1.2828x; 1.2828x over previous
import functools
import math

import jax
import jax.numpy as jnp
from jax import lax
from jax.experimental import pallas as pl
from jax.experimental.pallas import tpu as pltpu
from jax.experimental.pallas import tpu_sc as plsc

D = 16
NC = 2
NS = 16
NW = NC * NS
CHUNK = 1280
GSUB = 128
BB = CHUNK // 128
NSUB = CHUNK // GSUB
NGRP = CHUNK // D
LOGC = -0.5 * D * math.log(2.0 * math.pi)


def _mu_to_rowmajor(muT):
    n = muT.shape[1]
    s_blk = 12800
    r_blk = s_blk * D // 128
    grid = pl.cdiv(n, s_blk)

    def body(x_ref, o_ref):
        t = x_ref[...].T.reshape(r_blk, 8, D)
        for c in range(8):
            o_ref[:, c * D:(c + 1) * D] = t[:, c, :]

    out = pl.pallas_call(
        body,
        grid=(grid,),
        in_specs=[pl.BlockSpec((D, s_blk), lambda i: (0, i))],
        out_specs=pl.BlockSpec((r_blk, 128), lambda i: (i, 0)),
        out_shape=jax.ShapeDtypeStruct((n * D // 128, 128), jnp.float32),
    )(muT)
    return out.reshape(n, D)


@functools.partial(jax.jit, static_argnames=("m",))
def _log_prob_sc(zq, idx2d, mu, m):
    per_w = m // NW
    n_chunks = per_w // CHUNK

    mesh = plsc.VectorSubcoreMesh(core_axis_name="c", subcore_axis_name="s")

    @functools.partial(
        pl.kernel,
        out_type=jax.ShapeDtypeStruct((m,), jnp.float32),
        mesh=mesh,
        scratch_types=[
            pltpu.VMEM((2, CHUNK), jnp.int32),
            pltpu.VMEM((2, CHUNK, D), jnp.float32),
            pltpu.VMEM((2, 2, BB, 8, 128), jnp.float32),
            pltpu.VMEM((2, CHUNK), jnp.float32),
            pltpu.SemaphoreType.DMA,
            pltpu.SemaphoreType.DMA,
            pltpu.SemaphoreType.DMA,
            pltpu.SemaphoreType.DMA,
        ],
        compiler_params=pltpu.CompilerParams(
            use_tc_tiling_on_sc=False,
            needs_layout_passes=False,
        ),
    )
    def k(zq_hbm, idx_hbm, mu_hbm, out_hbm,
          idx_v, mu_v, z_v, out_v, sem_mu, sem_z, sem_i, sem_o):
        wid = lax.axis_index("s") * NC + lax.axis_index("c")
        row_base = wid * per_w

        def fire_idx(ci, b):
            cbase = row_base + ci * CHUNK
            pltpu.async_copy(
                idx_hbm.at[pl.ds(cbase, CHUNK)], idx_v.at[b], sem_i)

        def fire_inputs(ci, b):
            cbase = row_base + ci * CHUNK
            bb0 = cbase // 128
            for g in range(NSUB):
                pltpu.async_copy(
                    mu_hbm.at[idx_v.at[b, pl.ds(g * GSUB, GSUB)]],
                    mu_v.at[b, pl.ds(g * GSUB, GSUB)],
                    sem_mu,
                )
            for fg in range(2):
                pltpu.async_copy(
                    zq_hbm.at[fg, pl.ds(bb0, BB)], z_v.at[b, fg], sem_z)

        def wait_idx(b):
            pltpu.make_async_copy(
                idx_hbm.at[pl.ds(0, CHUNK)], idx_v.at[b], sem_i).wait()

        def wait_inputs(b):
            pltpu.make_async_copy(
                mu_hbm.at[pl.ds(0, CHUNK)], mu_v.at[b], sem_mu).wait()
            pltpu.make_async_copy(
                zq_hbm.at[:, pl.ds(0, BB)], z_v.at[b], sem_z).wait()

        def fire_out(ci, b):
            cbase = row_base + ci * CHUNK
            pltpu.async_copy(
                out_v.at[b], out_hbm.at[pl.ds(cbase, CHUNK)], sem_o)

        def wait_out(b):
            pltpu.make_async_copy(
                out_v.at[b], out_hbm.at[pl.ds(0, CHUNK)], sem_o).wait()

        def compute(b):
            lane = lax.iota(jnp.int32, D)

            @plsc.parallel_loop(0, NGRP, 1, unroll=2)
            def group_body(g):
                bbl = g // 8
                b0 = (g % 8) * D
                rows = g * D + lane
                acc = []
                for d in range(D):
                    zc = z_v[b, d // 8, bbl, d % 8, pl.ds(b0, D)]
                    mc = plsc.load_gather(
                        mu_v.at[b], [rows, jnp.full((D,), d, jnp.int32)]
                    )
                    t = zc - mc
                    sq = t * t
                    if d < 4:
                        acc.append(sq)
                    else:
                        acc[d % 4] = acc[d % 4] + sq
                tot = (acc[0] + acc[1]) + (acc[2] + acc[3])
                out_v[b, pl.ds(g * D, D)] = -0.5 * tot + LOGC

        fire_idx(0, 0)
        fire_idx(1, 1)
        wait_idx(0)
        fire_inputs(0, 0)

        @pl.loop(0, n_chunks, step=2)
        def chunk_loop(cc):
            for b in range(2):
                ci = cc + b
                nb = 1 - b

                @pl.when(ci + 1 < n_chunks)
                def _():
                    wait_idx(nb)
                    fire_inputs(ci + 1, nb)

                wait_inputs(b)

                @pl.when(ci + 2 < n_chunks)
                def _():
                    fire_idx(ci + 2, b)

                @pl.when(ci >= 2)
                def _():
                    wait_out(b)

                compute(b)
                fire_out(ci, b)

        wait_out(0)
        wait_out(1)

    return k(zq, idx2d, mu)


def kernel(z, sensor_idx, mu):
    m = z.shape[0]
    zq = z.reshape(m // 128, 128, 2, 8).transpose(2, 0, 3, 1)
    idx2d = sensor_idx.astype(jnp.int32)
    mu_rm = _mu_to_rowmajor(mu.T)
    return _log_prob_sc(zq, idx2d, mu_rm, m)
